# baseline (device time: 47207 ns/iter reference)
import jax
import jax.numpy as jnp
from jax import lax
from jax.experimental import pallas as pl
from jax.experimental.pallas import tpu as pltpu

N_DEV = 8

GROUPS = [
    dict(c0=0, cn=768, masks=(1, 3, 4), S=([2, 6, 5, 1], [7, 3], [4])),
    dict(c0=768, cn=640, masks=(3, 4, 1), S=([7, 6, 2, 3], [5, 4], [1])),
    dict(c0=1408, cn=640, masks=(4, 1, 3), S=([5, 6, 7, 4], [2, 1], [3])),
]
DOT_ORDER = [2, 7, 5, 6, 1, 3, 4, 0]
_J0 = (0, 4, 6)
QSCALES = (96.0, 64.0, 48.0)


def kernel(x, w_mat):
    m, _ = x.shape
    kk, n = w_mat.shape
    m_per = m // N_DEV

    def body(x_ref, w_ref, out_ref, acc,
             sq0, sq1, sq2, stq0, stq1, stq2, w16_ref,
             xbuf, obuf, xsem, osem, ss0, ss1, ss2, rs0, rs1, rs2):
        my = lax.axis_index("i")
        sq = (sq0, sq1, sq2)
        stq = (stq0, stq1, stq2)
        ssems = (ss0, ss1, ss2)
        rsems = (rs0, rs1, rs2)

        def load_x(mo, par):
            pltpu.make_async_copy(
                x_ref.at[pl.ds((my ^ mo) * m_per, m_per), :],
                xbuf.at[par], xsem.at[par],
            ).start()

        def wait_x(par):
            pltpu.make_async_copy(
                x_ref.at[pl.ds(0, m_per), :], xbuf.at[par], xsem.at[par],
            ).wait()

        load_x(DOT_ORDER[0], 0)

        barrier_sem = pltpu.get_barrier_semaphore()
        for mk in (1, 3, 4):
            pl.semaphore_signal(
                barrier_sem, inc=1,
                device_id=(my ^ mk,), device_id_type=pl.DeviceIdType.MESH,
            )
        pl.semaphore_wait(barrier_sem, 3)

        w16_ref[...] = w_ref[...].astype(jnp.bfloat16)

        def mk_rdma(g, p, t):
            G = GROUPS[g]
            j = _J0[p] + t
            return pltpu.make_async_remote_copy(
                src_ref=sq[g].at[j],
                dst_ref=stq[g].at[j],
                send_sem=ssems[g].at[j],
                recv_sem=rsems[g].at[j],
                device_id=(my ^ G["masks"][p],),
                device_id_type=pl.DeviceIdType.MESH,
            )

        def quantize_val(val, g, p, t):
            sq[g][_J0[p] + t] = jnp.clip(
                jnp.round(val * QSCALES[p]), -127.0, 127.0,
            ).astype(jnp.int8)

        ptr = [0, 0, 0]
        computed = set()

        def dot_step(i):
            mo = DOT_ORDER[i]
            wait_x(i % 2)
            if i + 1 < N_DEV:
                load_x(DOT_ORDER[i + 1], (i + 1) % 2)
            xs = xbuf[i % 2].astype(jnp.bfloat16)
            val = jnp.dot(xs, w16_ref[...], preferred_element_type=jnp.float32)
            acc[mo, :, :] = val.astype(jnp.bfloat16)
            computed.add(mo)
            for g in range(3):
                G = GROUPS[g]
                S0 = G["S"][0]
                if mo in S0:
                    quantize_val(val[:, G["c0"]:G["c0"] + G["cn"]],
                                 g, 0, S0.index(mo))
                while ptr[g] < 4 and S0[ptr[g]] in computed:
                    mk_rdma(g, 0, ptr[g]).start()
                    ptr[g] += 1

        for i in range(4):
            dot_step(i)

        def recv_add(g, p, t, quant_next=None):
            G = GROUPS[g]
            mo = G["S"][p][t] ^ G["masks"][p]
            mk_rdma(g, p, t).wait_recv()
            cols = pl.ds(G["c0"], G["cn"])
            contrib = stq[g][_J0[p] + t].astype(jnp.bfloat16) * jnp.bfloat16(
                1.0 / QSCALES[p])
            s = acc[mo, :, cols] + contrib
            acc[mo, :, cols] = s
            if quant_next is not None:
                pq, tq = quant_next
                quantize_val(s.astype(jnp.float32), g, pq, tq)

        for t in range(4):
            dot_step(4 + t)
            for g in (1, 2, 0):
                recv_add(g, 0, t, quant_next=(1, 1 - t) if t < 2 else None)
                if t == 1:
                    mk_rdma(g, 1, 0).start()
                    mk_rdma(g, 1, 1).start()
        for t in range(2):
            for g in (1, 2, 0):
                recv_add(g, 1, t, quant_next=(2, 0) if t == 0 else None)
                if t == 0:
                    mk_rdma(g, 2, 0).start()
        for g in (1, 2, 0):
            recv_add(g, 2, 0)
            G = GROUPS[g]
            cols = pl.ds(G["c0"], G["cn"])
            y = acc[0, :, cols].astype(jnp.float32)
            obuf[:, cols] = y * jax.nn.sigmoid(y)
            pltpu.make_async_copy(
                obuf.at[:, cols], out_ref.at[:, cols], osem.at[g],
            ).start()

        for g in range(3):
            G = GROUPS[g]
            cols = pl.ds(G["c0"], G["cn"])
            pltpu.make_async_copy(
                obuf.at[:, cols], out_ref.at[:, cols], osem.at[g],
            ).wait()
            for p in range(3):
                for t in range(len(GROUPS[g]["S"][p])):
                    mk_rdma(g, p, t).wait_send()

    return pl.pallas_call(
        body,
        out_shape=jax.ShapeDtypeStruct((m_per, n), jnp.float32),
        in_specs=[
            pl.BlockSpec(memory_space=pltpu.MemorySpace.HBM),
            pl.BlockSpec(memory_space=pltpu.MemorySpace.VMEM),
        ],
        out_specs=pl.BlockSpec(memory_space=pltpu.MemorySpace.HBM),
        scratch_shapes=[
            pltpu.VMEM((N_DEV, m_per, n), jnp.bfloat16),
            pltpu.VMEM((7, m_per, 768), jnp.int8),
            pltpu.VMEM((7, m_per, 640), jnp.int8),
            pltpu.VMEM((7, m_per, 640), jnp.int8),
            pltpu.VMEM((7, m_per, 768), jnp.int8),
            pltpu.VMEM((7, m_per, 640), jnp.int8),
            pltpu.VMEM((7, m_per, 640), jnp.int8),
            pltpu.VMEM((kk, n), jnp.bfloat16),
            pltpu.VMEM((2, m_per, kk), jnp.float32),
            pltpu.VMEM((m_per, n), jnp.float32),
            pltpu.SemaphoreType.DMA((2,)),
            pltpu.SemaphoreType.DMA((3,)),
            pltpu.SemaphoreType.DMA((7,)),
            pltpu.SemaphoreType.DMA((7,)),
            pltpu.SemaphoreType.DMA((7,)),
            pltpu.SemaphoreType.DMA((7,)),
            pltpu.SemaphoreType.DMA((7,)),
            pltpu.SemaphoreType.DMA((7,)),
        ],
        compiler_params=pltpu.CompilerParams(
            collective_id=0, vmem_limit_bytes=64 * 1024 * 1024),
    )(x, w_mat)


# device time: 44488 ns/iter; 1.0611x vs baseline; 1.0611x over previous
import jax
import jax.numpy as jnp
from jax import lax
from jax.experimental import pallas as pl
from jax.experimental.pallas import tpu as pltpu

N_DEV = 8

GROUPS = [
    dict(c0=0, cn=768, masks=(1, 3, 4), S=([2, 6, 5, 1], [7, 3], [4])),
    dict(c0=768, cn=640, masks=(3, 4, 1), S=([7, 6, 2, 3], [5, 4], [1])),
    dict(c0=1408, cn=640, masks=(4, 1, 3), S=([5, 6, 7, 4], [2, 1], [3])),
]
DOT_ORDER = [2, 7, 5, 6, 1, 3, 4, 0]
_J0 = (0, 4, 6)
QSCALES = (96.0, 64.0, 48.0)


def kernel(x, w_mat):
    m, _ = x.shape
    kk, n = w_mat.shape
    m_per = m // N_DEV

    def body(x_ref, w_ref, out_ref, acc,
             sq0, sq1, sq2, stq0, stq1, stq2, w16_ref,
             xbuf, obuf, xsem, osem, ss0, ss1, ss2, rs0, rs1, rs2):
        my = lax.axis_index("i")
        sq = (sq0, sq1, sq2)
        stq = (stq0, stq1, stq2)
        ssems = (ss0, ss1, ss2)
        rsems = (rs0, rs1, rs2)

        def load_x(mo, par):
            pltpu.make_async_copy(
                x_ref.at[pl.ds((my ^ mo) * m_per, m_per), :],
                xbuf.at[par], xsem.at[par],
            ).start()

        def wait_x(par):
            pltpu.make_async_copy(
                x_ref.at[pl.ds(0, m_per), :], xbuf.at[par], xsem.at[par],
            ).wait()

        load_x(DOT_ORDER[0], 0)

        barrier_sem = pltpu.get_barrier_semaphore()
        for mk in (1, 3, 4):
            pl.semaphore_signal(
                barrier_sem, inc=1,
                device_id=(my ^ mk,), device_id_type=pl.DeviceIdType.MESH,
            )
        pl.semaphore_wait(barrier_sem, 3)

        w16_ref[...] = w_ref[...].astype(jnp.bfloat16)

        def mk_rdma(g, p, t):
            G = GROUPS[g]
            j = _J0[p] + t
            return pltpu.make_async_remote_copy(
                src_ref=sq[g].at[j],
                dst_ref=stq[g].at[j],
                send_sem=ssems[g].at[j],
                recv_sem=rsems[g].at[j],
                device_id=(my ^ G["masks"][p],),
                device_id_type=pl.DeviceIdType.MESH,
            )

        def quantize_val(val, g, p, t):
            sq[g][_J0[p] + t] = jnp.clip(
                jnp.round(val * QSCALES[p]), -127.0, 127.0,
            ).astype(jnp.int8)

        ptr = [0, 0, 0]
        computed = set()

        def dot_step(i):
            mo = DOT_ORDER[i]
            wait_x(i % 2)
            if i + 1 < N_DEV:
                load_x(DOT_ORDER[i + 1], (i + 1) % 2)
            xs = xbuf[i % 2].astype(jnp.bfloat16)
            val = jnp.dot(xs, w16_ref[...], preferred_element_type=jnp.float32)
            acc[mo, :, :] = val.astype(jnp.bfloat16)
            computed.add(mo)
            for g in range(3):
                G = GROUPS[g]
                S0 = G["S"][0]
                if mo in S0:
                    quantize_val(val[:, G["c0"]:G["c0"] + G["cn"]],
                                 g, 0, S0.index(mo))
                while ptr[g] < 4 and S0[ptr[g]] in computed:
                    mk_rdma(g, 0, ptr[g]).start()
                    ptr[g] += 1

        for i in range(N_DEV - 1):
            dot_step(i)

        def recv_add(g, p, t, quant_next=None):
            G = GROUPS[g]
            mo = G["S"][p][t] ^ G["masks"][p]
            mk_rdma(g, p, t).wait_recv()
            cols = pl.ds(G["c0"], G["cn"])
            contrib = stq[g][_J0[p] + t].astype(jnp.bfloat16) * jnp.bfloat16(
                1.0 / QSCALES[p])
            s = acc[mo, :, cols] + contrib
            acc[mo, :, cols] = s
            if quant_next is not None:
                pq, tq = quant_next
                quantize_val(s.astype(jnp.float32), g, pq, tq)

        for t in range(4):
            if t == 2:
                dot_step(N_DEV - 1)
            for g in (1, 2, 0):
                recv_add(g, 0, t, quant_next=(1, 1 - t) if t < 2 else None)
                if t == 1:
                    mk_rdma(g, 1, 0).start()
                    mk_rdma(g, 1, 1).start()
        for t in range(2):
            for g in (1, 2, 0):
                recv_add(g, 1, t, quant_next=(2, 0) if t == 0 else None)
                if t == 0:
                    mk_rdma(g, 2, 0).start()
        for g in (1, 2, 0):
            recv_add(g, 2, 0)
            G = GROUPS[g]
            cols = pl.ds(G["c0"], G["cn"])
            y = acc[0, :, cols].astype(jnp.float32)
            obuf[:, cols] = y * jax.nn.sigmoid(y)
            pltpu.make_async_copy(
                obuf.at[:, cols], out_ref.at[:, cols], osem.at[g],
            ).start()

        for g in range(3):
            G = GROUPS[g]
            cols = pl.ds(G["c0"], G["cn"])
            pltpu.make_async_copy(
                obuf.at[:, cols], out_ref.at[:, cols], osem.at[g],
            ).wait()
            for p in range(3):
                for t in range(len(GROUPS[g]["S"][p])):
                    mk_rdma(g, p, t).wait_send()

    return pl.pallas_call(
        body,
        out_shape=jax.ShapeDtypeStruct((m_per, n), jnp.float32),
        in_specs=[
            pl.BlockSpec(memory_space=pltpu.MemorySpace.HBM),
            pl.BlockSpec(memory_space=pltpu.MemorySpace.VMEM),
        ],
        out_specs=pl.BlockSpec(memory_space=pltpu.MemorySpace.HBM),
        scratch_shapes=[
            pltpu.VMEM((N_DEV, m_per, n), jnp.bfloat16),
            pltpu.VMEM((7, m_per, 768), jnp.int8),
            pltpu.VMEM((7, m_per, 640), jnp.int8),
            pltpu.VMEM((7, m_per, 640), jnp.int8),
            pltpu.VMEM((7, m_per, 768), jnp.int8),
            pltpu.VMEM((7, m_per, 640), jnp.int8),
            pltpu.VMEM((7, m_per, 640), jnp.int8),
            pltpu.VMEM((kk, n), jnp.bfloat16),
            pltpu.VMEM((2, m_per, kk), jnp.float32),
            pltpu.VMEM((m_per, n), jnp.float32),
            pltpu.SemaphoreType.DMA((2,)),
            pltpu.SemaphoreType.DMA((3,)),
            pltpu.SemaphoreType.DMA((7,)),
            pltpu.SemaphoreType.DMA((7,)),
            pltpu.SemaphoreType.DMA((7,)),
            pltpu.SemaphoreType.DMA((7,)),
            pltpu.SemaphoreType.DMA((7,)),
            pltpu.SemaphoreType.DMA((7,)),
        ],
        compiler_params=pltpu.CompilerParams(
            collective_id=0, vmem_limit_bytes=64 * 1024 * 1024),
    )(x, w_mat)
